# baseline (speedup 1.0000x reference)
import functools

import jax
import jax.numpy as jnp
from jax import lax
from jax.experimental import pallas as pl
from jax.experimental.pallas import tpu as pltpu
from jax.experimental.pallas import tpu_sc as plsc

_NUM_CORES = 2


def _make_sc_copy(L, D, dtype):
    rows_per_c = L // _NUM_CORES
    mesh = plsc.ScalarSubcoreMesh(axis_name="c", num_cores=_NUM_CORES)

    @functools.partial(
        pl.kernel,
        mesh=mesh,
        out_type=jax.ShapeDtypeStruct((L, D), dtype),
        scratch_types=[pltpu.VMEM_SHARED((rows_per_c, D), dtype)],
    )
    def sc_copy(emb_hbm, out_hbm, buf):
        cid = lax.axis_index("c")
        base = cid * rows_per_c
        pltpu.sync_copy(emb_hbm.at[pl.ds(base, rows_per_c)], buf)
        pltpu.sync_copy(buf, out_hbm.at[pl.ds(base, rows_per_c)])

    return sc_copy


def kernel(x, emb):
    L = x.shape[1]
    D = emb.shape[1]
    out = _make_sc_copy(L, D, emb.dtype)(emb)
    return out[None]


# speedup vs baseline: 1.1459x; 1.1459x over previous
import functools

import jax
import jax.numpy as jnp
from jax import lax
from jax.experimental import pallas as pl
from jax.experimental.pallas import tpu as pltpu
from jax.experimental.pallas import tpu_sc as plsc

_NUM_CORES = 2
_NUM_SUBCORES = 16
_NUM_WORKERS = _NUM_CORES * _NUM_SUBCORES
_NUM_CHUNKS = 4


def _make_sc_copy(L, D, dtype):
    rows_per_w = L // _NUM_WORKERS
    crows = rows_per_w // _NUM_CHUNKS
    mesh = plsc.VectorSubcoreMesh(core_axis_name="c", subcore_axis_name="s")

    @functools.partial(
        pl.kernel,
        mesh=mesh,
        out_type=jax.ShapeDtypeStruct((L, D), dtype),
        scratch_types=(
            [pltpu.VMEM((rows_per_w, D), dtype)]
            + [pltpu.SemaphoreType.DMA] * (2 * _NUM_CHUNKS)
        ),
    )
    def sc_copy(emb_hbm, out_hbm, buf, *sems):
        in_sems = sems[:_NUM_CHUNKS]
        out_sems = sems[_NUM_CHUNKS:]
        wid = lax.axis_index("s") * _NUM_CORES + lax.axis_index("c")
        base = wid * rows_per_w
        ins = []
        for j in range(_NUM_CHUNKS):
            ins.append(
                pltpu.async_copy(
                    emb_hbm.at[pl.ds(base + j * crows, crows)],
                    buf.at[pl.ds(j * crows, crows)],
                    in_sems[j],
                )
            )
        outs = []
        for j in range(_NUM_CHUNKS):
            ins[j].wait()
            outs.append(
                pltpu.async_copy(
                    buf.at[pl.ds(j * crows, crows)],
                    out_hbm.at[pl.ds(base + j * crows, crows)],
                    out_sems[j],
                )
            )
        for o in outs:
            o.wait()

    return sc_copy


def kernel(x, emb):
    L = x.shape[1]
    D = emb.shape[1]
    out = _make_sc_copy(L, D, emb.dtype)(emb)
    return out[None]


# speedup vs baseline: 1.1534x; 1.0066x over previous
import functools

import jax
import jax.numpy as jnp
from jax import lax
from jax.experimental import pallas as pl
from jax.experimental.pallas import tpu as pltpu
from jax.experimental.pallas import tpu_sc as plsc

_NUM_CORES = 2
_NUM_SUBCORES = 16
_NUM_WORKERS = _NUM_CORES * _NUM_SUBCORES


def _make_sc_copy(L, D, dtype):
    rows_per_w = L // _NUM_WORKERS
    mesh = plsc.VectorSubcoreMesh(core_axis_name="c", subcore_axis_name="s")

    @functools.partial(
        pl.kernel,
        mesh=mesh,
        out_type=jax.ShapeDtypeStruct((L, D), dtype),
        scratch_types=[pltpu.VMEM((rows_per_w, D), dtype)],
    )
    def sc_copy(emb_hbm, out_hbm, buf):
        wid = lax.axis_index("s") * _NUM_CORES + lax.axis_index("c")
        base = wid * rows_per_w
        pltpu.sync_copy(emb_hbm.at[pl.ds(base, rows_per_w)], buf)
        pltpu.sync_copy(buf, out_hbm.at[pl.ds(base, rows_per_w)])

    return sc_copy


def kernel(x, emb):
    L = x.shape[1]
    D = emb.shape[1]
    out = _make_sc_copy(L, D, emb.dtype)(emb)
    return out[None]
